# Initial kernel scaffold; baseline (speedup 1.0000x reference)
import jax
import jax.numpy as jnp
from jax.experimental import pallas as pl


def kernel(x, edge_index, batch, W1, b1, W2, b2, W3, b3, Wg1, bg1, Wg2, bg2):
    raise NotImplementedError("write your pallas kernel here")


# speedup vs baseline: 4.0154x; 4.0154x over previous
import functools

import jax
import jax.numpy as jnp
from jax import lax
from jax.experimental import pallas as pl
from jax.experimental.pallas import tpu as pltpu
from jax.experimental.pallas import tpu_sc as plsc

N = 10000
NP = 10240
E = 160000
G = 64
NC = 2
NS = 16
NW = NC * NS
CH = 128
NCHUNK = 40
EPW = NCHUNK * CH
EPAD = EPW * NW
DUMMY = 10008
RPT = NP // NS
R_BLK = 1280
P_BLK = 640

_sc_mesh = plsc.VectorSubcoreMesh(
    core_axis_name="c", subcore_axis_name="s", num_cores=NC, num_subcores=NS
)


def _make_agg(S):
    @functools.partial(
        pl.kernel,
        out_type=jax.ShapeDtypeStruct((NC * S * NP, 128), jnp.float32),
        mesh=_sc_mesh,
        scratch_types=[
            pltpu.VMEM((NCHUNK, CH), jnp.int32),
            pltpu.VMEM((NCHUNK, CH), jnp.int32),
            pltpu.VMEM((CH, 128), jnp.float32),
            pltpu.VMEM_SHARED((NP, 128), jnp.float32),
            pltpu.SemaphoreType.DMA,
        ],
    )
    def agg(s_hbm, src_hbm, dst_hbm, out_hbm, srcv, dstv, gbuf, acc, sem):
        c = lax.axis_index("c")
        t = lax.axis_index("s")
        w = c * NS + t
        pltpu.sync_copy(dst_hbm.at[pl.ds(w * NCHUNK, NCHUNK)], dstv)
        for sl in range(S):
            pltpu.sync_copy(
                src_hbm.at[pl.ds((sl * NW + w) * NCHUNK, NCHUNK)], srcv
            )
            init = jnp.where(c == 0, sl * NP, S * NP) + t * RPT
            pltpu.sync_copy(
                s_hbm.at[pl.ds(init, RPT)], acc.at[pl.ds(t * RPT, RPT)]
            )
            plsc.subcore_barrier()

            def ebody(j, carry):
                pltpu.async_copy(s_hbm.at[srcv.at[j]], gbuf, sem).wait()
                pltpu.sync_copy(gbuf, acc.at[dstv.at[j]], add=True)
                return carry

            lax.fori_loop(0, NCHUNK, ebody, 0)
            plsc.subcore_barrier()
            out = (c * S + sl) * NP + t * RPT
            pltpu.sync_copy(
                acc.at[pl.ds(t * RPT, RPT)], out_hbm.at[pl.ds(out, RPT)]
            )
            if sl + 1 < S:
                plsc.subcore_barrier()

    return agg


_agg1 = _make_agg(1)
_agg2 = _make_agg(2)
_agg4 = _make_agg(4)


def _prescale_body(x_ref, dp_ref, out_ref):
    d = lax.rsqrt(dp_ref[0][:, 0:1] + dp_ref[1][:, 0:1])
    sx = x_ref[...] * d
    out_ref[0] = sx[:, :128]
    out_ref[1] = sx[:, 128:]


def _prescale(x, degp):
    return pl.pallas_call(
        _prescale_body,
        grid=(NP // R_BLK,),
        in_specs=[
            pl.BlockSpec((R_BLK, 256), lambda i: (i, 0)),
            pl.BlockSpec((2, R_BLK, 128), lambda i: (0, i, 0)),
        ],
        out_specs=pl.BlockSpec((2, R_BLK, 128), lambda i: (0, i, 0)),
        out_shape=jax.ShapeDtypeStruct((2, NP, 128), jnp.float32),
    )(x, degp)


def _make_layer(Si, Dout, post_scale, out_sliced):
    So = Dout // 128

    def body(a_ref, dp_ref, w_ref, b_ref, out_ref):
        d = lax.rsqrt(dp_ref[0][:, 0:1] + dp_ref[1][:, 0:1])
        acc = None
        for ci in range(Si):
            xs = (a_ref[ci] + a_ref[Si + ci]) * d
            p = jnp.dot(
                xs,
                w_ref[ci * 128 : (ci + 1) * 128, :],
                preferred_element_type=jnp.float32,
            )
            acc = p if acc is None else acc + p
        z = jnp.maximum(acc + b_ref[0], 0.0)
        if post_scale:
            z = z * d
        if out_sliced:
            for j in range(So):
                out_ref[j] = z[:, j * 128 : (j + 1) * 128]
        else:
            out_ref[...] = z

    if out_sliced:
        out_specs = pl.BlockSpec((So, R_BLK, 128), lambda i: (0, i, 0))
        out_shape = jax.ShapeDtypeStruct((So, NP, 128), jnp.float32)
    else:
        out_specs = pl.BlockSpec((R_BLK, Dout), lambda i: (i, 0))
        out_shape = jax.ShapeDtypeStruct((NP, Dout), jnp.float32)

    def run(a3d, degp, W, b2d):
        return pl.pallas_call(
            body,
            grid=(NP // R_BLK,),
            in_specs=[
                pl.BlockSpec((2 * Si, R_BLK, 128), lambda i: (0, i, 0)),
                pl.BlockSpec((2, R_BLK, 128), lambda i: (0, i, 0)),
                pl.BlockSpec((Si * 128, Dout), lambda i: (0, 0)),
                pl.BlockSpec((1, Dout), lambda i: (0, 0)),
            ],
            out_specs=out_specs,
            out_shape=out_shape,
        )(a3d, degp, W, b2d)

    return run


_layer1 = _make_layer(2, 256, True, True)
_layer2 = _make_layer(2, 512, True, True)
_layer3 = _make_layer(4, 1024, False, False)


def _pool_body(h_ref, b_ref, out_ref):
    i = pl.program_id(0)

    @pl.when(i == 0)
    def _():
        out_ref[...] = jnp.full((G, 1024), -jnp.inf, jnp.float32)

    h = h_ref[...]
    bcol = b_ref[:, 0:1]
    rows = []
    for g in range(G):
        m = jnp.where(bcol == g, 0.0, -jnp.inf)
        rows.append(jnp.max(h + m, axis=0, keepdims=True))
    out_ref[...] = jnp.maximum(out_ref[...], jnp.concatenate(rows, axis=0))


def _pool(h3, batchb):
    return pl.pallas_call(
        _pool_body,
        grid=(NP // P_BLK,),
        in_specs=[
            pl.BlockSpec((P_BLK, 1024), lambda i: (i, 0)),
            pl.BlockSpec((P_BLK, 128), lambda i: (i, 0)),
        ],
        out_specs=pl.BlockSpec((G, 1024), lambda i: (0, 0)),
        out_shape=jax.ShapeDtypeStruct((G, 1024), jnp.float32),
    )(h3, batchb)


def _fc_body(p_ref, wg1_ref, bg1_ref, wg2_ref, bg2_ref, out_ref):
    gmat = jnp.maximum(
        jnp.dot(p_ref[...], wg1_ref[...], preferred_element_type=jnp.float32)
        + bg1_ref[0],
        0.0,
    )
    out_ref[...] = (
        jnp.dot(gmat, wg2_ref[...], preferred_element_type=jnp.float32) + bg2_ref[0]
    )


def _fc(pooled, Wg1, bg1_2d, Wg2, bg2_2d):
    return pl.pallas_call(
        _fc_body,
        out_shape=jax.ShapeDtypeStruct((G, 128), jnp.float32),
    )(pooled, Wg1, bg1_2d, Wg2, bg2_2d)


def _edge_tables(edge_index):
    pad = EPAD - E
    src = jnp.concatenate([edge_index[0], jnp.zeros((pad,), jnp.int32)])
    dst = jnp.concatenate([edge_index[1], jnp.full((pad,), DUMMY, jnp.int32)])
    src2d = src.reshape(NW * NCHUNK, CH)
    dst2d = dst.reshape(NW * NCHUNK, CH)
    offs = {}
    for S in (1, 2, 4):
        o = (jnp.arange(S, dtype=jnp.int32) * NP)[:, None, None]
        offs[S] = (src2d[None] + o).reshape(S * NW * NCHUNK, CH)
    return offs, dst2d


def _with_zeros(s_flat):
    return jnp.concatenate([s_flat, jnp.zeros((NP, 128), jnp.float32)], axis=0)


def kernel(x, edge_index, batch, W1, b1, W2, b2, W3, b3, Wg1, bg1, Wg2, bg2):
    offs, dst2d = _edge_tables(edge_index)
    xp = jnp.pad(x, ((0, NP - N), (0, 0)))
    batchb = jnp.broadcast_to(
        jnp.concatenate([batch, jnp.full((NP - N,), G, jnp.int32)])[:, None],
        (NP, 128),
    )

    ones = jnp.ones((NP, 128), jnp.float32)
    degp = _agg1(_with_zeros(ones), offs[1], dst2d).reshape(2, NP, 128)

    s0 = _prescale(xp, degp)
    a1 = _agg2(_with_zeros(s0.reshape(2 * NP, 128)), offs[2], dst2d)
    s1 = _layer1(a1.reshape(4, NP, 128), degp, W1, b1.reshape(1, -1))
    a2 = _agg2(_with_zeros(s1.reshape(2 * NP, 128)), offs[2], dst2d)
    s2 = _layer2(a2.reshape(4, NP, 128), degp, W2, b2.reshape(1, -1))
    a3 = _agg4(_with_zeros(s2.reshape(4 * NP, 128)), offs[4], dst2d)
    h3 = _layer3(a3.reshape(8, NP, 128), degp, W3, b3.reshape(1, -1))
    pooled = _pool(h3, batchb)
    out = _fc(pooled, Wg1, bg1.reshape(1, -1), Wg2, bg2.reshape(1, -1))
    return out
